# baseline (speedup 1.0000x reference)
import functools
import math

import jax
import jax.numpy as jnp
from jax import lax
from jax.experimental import pallas as pl
from jax.experimental.pallas import tpu as pltpu
from jax.experimental.pallas import tpu_sc as plsc

D_MODEL = 64
SCALE = math.sqrt(D_MODEL)
NUM_CORES = 2
NUM_SUBCORES = 16
NUM_WORKERS = NUM_CORES * NUM_SUBCORES
ROWS_B = 8


def _unflatten_scale(yf, b, l):
  grid = b // (2 * ROWS_B)
  blk = ROWS_B * l

  def body(y_ref, o_ref):
    y = y_ref[...] * SCALE
    o_ref[pl.ds(0, ROWS_B)] = y[:, :D_MODEL].reshape(ROWS_B, l, D_MODEL)
    o_ref[pl.ds(ROWS_B, ROWS_B)] = y[:, D_MODEL:].reshape(ROWS_B, l, D_MODEL)

  return pl.pallas_call(
      body,
      grid=(grid,),
      in_specs=[pl.BlockSpec((blk, 128), lambda i: (i, 0))],
      out_specs=pl.BlockSpec((2 * ROWS_B, l, D_MODEL), lambda i: (i, 0, 0)),
      out_shape=jax.ShapeDtypeStruct((b, l, D_MODEL), jnp.float32),
  )(yf)


def _sc_gather(x1, x2, emb, b, l):
  n_half = b * l // 2
  xrows_per_w = b // NUM_WORKERS
  half_w = NUM_WORKERS // 2
  n_pairs = xrows_per_w // 2
  w2 = l - 128
  w2_off = 128 - w2

  mesh = plsc.VectorSubcoreMesh(
      core_axis_name="c", subcore_axis_name="s",
      num_cores=NUM_CORES, num_subcores=NUM_SUBCORES)

  @functools.partial(
      pl.kernel,
      mesh=mesh,
      out_type=jax.ShapeDtypeStruct((n_half, 128), jnp.float32),
      compiler_params=pltpu.CompilerParams(use_tc_tiling_on_sc=False),
      scratch_types=[
          pltpu.VMEM((xrows_per_w, 128), jnp.int32),
          pltpu.VMEM((xrows_per_w, 128), jnp.int32),
          pltpu.VMEM((l, D_MODEL), jnp.float32),
          pltpu.VMEM((l, D_MODEL), jnp.float32),
          pltpu.SemaphoreType.DMA,
          pltpu.SemaphoreType.DMA,
          pltpu.SemaphoreType.DMA,
          pltpu.SemaphoreType.DMA,
      ],
  )
  def k(x1_hbm, x2_hbm, emb_hbm, yf_hbm, v1, v2, rows0, rows1,
        g0, g1, s0, s1):
    wid = lax.axis_index("s") * NUM_CORES + lax.axis_index("c")
    xrow0 = wid * xrows_per_w
    rows = (rows0, rows1)
    gsem = (g0, g1)
    ssem = (s0, s1)

    pltpu.sync_copy(x1_hbm.at[pl.ds(xrow0, xrows_per_w)], v1)
    pltpu.sync_copy(x2_hbm.at[pl.ds(xrow0, xrows_per_w)], v2)

    def gathers(r, bb):
      return [
          pltpu.make_async_copy(
              emb_hbm.at[v1.at[r]],
              rows[bb].at[pl.ds(0, 128)], gsem[bb]),
          pltpu.make_async_copy(
              emb_hbm.at[v2.at[r, pl.ds(w2_off, w2)]],
              rows[bb].at[pl.ds(128, w2)], gsem[bb]),
      ]

    def start_gathers(r, bb):
      for g in gathers(r, bb):
        g.start()

    def wait_gathers(r, bb):
      for g in gathers(r, bb):
        g.wait()

    def store(r, bb):
      rg = xrow0 + r
      blkq = rg // 16
      pos = rg % 16
      col = jnp.where(pos < 8, 0, D_MODEL)
      qrow = blkq * (16 * l // 2) + jnp.where(pos < 8, pos, pos - 8) * l
      return pltpu.make_async_copy(
          rows[bb],
          yf_hbm.at[pl.ds(qrow, l), pl.ds(col, D_MODEL)],
          ssem[bb])

    start_gathers(0, 0)

    def pair_body(i, carry):
      for bb in range(2):
        r = 2 * i + bb
        other = 1 - bb

        if bb == 0:
          @pl.when(i > 0)
          def _():
            store(r - 1, other).wait()
          start_gathers(r + 1, other)
        else:
          store(r - 1, other).wait()
          @pl.when(i < n_pairs - 1)
          def _():
            start_gathers(r + 1, other)

        wait_gathers(r, bb)
        store(r, bb).start()
      return carry

    lax.fori_loop(0, n_pairs, pair_body, 0)
    store(xrows_per_w - 1, 1).wait()

  return k(x1, x2, emb)


@jax.jit
def _embed(x, emb):
  b, l = x.shape
  x1 = x[:, :128]
  x2 = x[:, l - 128:]
  yf = _sc_gather(x1, x2, emb, b, l)
  return _unflatten_scale(yf, b, l)


def kernel(x, emb):
  return _embed(x.astype(jnp.int32), emb)
